# baseline (speedup 1.0000x reference)
import functools

import jax
import jax.numpy as jnp
from jax import lax
from jax.experimental import pallas as pl
from jax.experimental.pallas import tpu as pltpu
from jax.experimental.pallas import tpu_sc as plsc

_N_ITEMS = 1024
_N_SAMPLES = 65536
_BATCH = 64

_L = 16
_NC, _NS = 2, 16
_NW = _NC * _NS
_TR = 8
_NQ = 4
_QW = _N_SAMPLES // _NQ
_WC = 4096
_NCH = _QW // _WC
_DEPTH = 3
_SEL_CHUNKS = _N_ITEMS // _L


def _argmax_rows(sel_ref, nrows):
    offs = lax.iota(jnp.int32, _L)

    def step(c, carry):
        out = []
        for r, (maxv, idxv) in enumerate(carry):
            vals = sel_ref[r, pl.ds(c * _L, _L)]
            take = vals > maxv
            out.append((jnp.where(take, vals, maxv),
                        jnp.where(take, c * _L + offs, idxv)))
        return tuple(out)

    init = tuple(
        (jnp.full((_L,), -jnp.inf, jnp.float32), jnp.zeros((_L,), jnp.int32))
        for _ in range(nrows)
    )
    carry = lax.fori_loop(0, _SEL_CHUNKS, step, init)
    dnums = lax.GatherDimensionNumbers(
        offset_dims=(), collapsed_slice_dims=(0,), start_index_map=(0,)
    )
    shuf = lambda v, perm: lax.gather(
        v, perm[:, None], dnums, (1,),
        mode=lax.GatherScatterMode.PROMISE_IN_BOUNDS,
    )
    merged = jnp.zeros((_L,), jnp.int32)
    for r in range(nrows):
        maxv, idxv = carry[r]
        for k in (1, 2, 4, 8):
            perm = offs ^ k
            ov = shuf(maxv, perm)
            oi = shuf(idxv, perm)
            take = (ov > maxv) | ((ov == maxv) & (oi < idxv))
            maxv = jnp.where(take, ov, maxv)
            idxv = jnp.where(take, oi, idxv)
        merged = jnp.where(offs == r, idxv, merged)
    return merged


def _body(sel_hbm, items_hbm, out_hbm, sel_v, idx_v, rows_v, gsem, ssem):
    wid = lax.axis_index("s") * _NC + lax.axis_index("c")
    t = wid // _NQ
    q = wid % _NQ
    r0 = t * _TR
    c0 = q * _QW

    pltpu.sync_copy(sel_hbm.at[pl.ds(r0, _TR)], sel_v)
    idx_v[...] = _argmax_rows(sel_v, _TR)

    def gather(g, buf):
        return pltpu.async_copy(
            items_hbm.at[idx_v.at[pl.ds(0, _TR)], pl.ds(c0 + g * _WC, _WC)],
            rows_v.at[buf],
            gsem.at[buf],
        )

    def scatter(g, buf):
        return pltpu.async_copy(
            rows_v.at[buf],
            out_hbm.at[pl.ds(r0, _TR), pl.ds(c0 + g * _WC, _WC)],
            ssem.at[buf],
        )

    gath = {}
    scats = {}
    for g in range(min(_DEPTH, _NCH)):
        gath[g] = gather(g, g % _DEPTH)
    for g in range(_NCH):
        buf = g % _DEPTH
        gath[g].wait()
        scats[g] = scatter(g, buf)
        nxt = g + _DEPTH
        if nxt < _NCH:
            if nxt - _DEPTH in scats:
                scats[nxt - _DEPTH].wait()
                del scats[nxt - _DEPTH]
            gath[nxt] = gather(nxt, nxt % _DEPTH)
    for g in sorted(scats):
        scats[g].wait()


@jax.jit
def kernel(selections, items):
    call = functools.partial(
        pl.kernel,
        out_type=jax.ShapeDtypeStruct((_BATCH, _N_SAMPLES), jnp.float32),
        mesh=plsc.VectorSubcoreMesh(core_axis_name="c", subcore_axis_name="s"),
        compiler_params=pltpu.CompilerParams(use_tc_tiling_on_sc=True),
        scratch_types=[
            pltpu.VMEM((_TR, _N_ITEMS), jnp.float32),
            pltpu.VMEM((_L,), jnp.int32),
            pltpu.VMEM((_DEPTH, _TR, _WC), jnp.float32),
            pltpu.SemaphoreType.DMA((_DEPTH,)),
            pltpu.SemaphoreType.DMA((_DEPTH,)),
        ],
    )(_body)
    return call(selections, items)
